# baseline (speedup 1.0000x reference)
import jax
import jax.numpy as jnp
from jax import lax
from jax.experimental import pallas as pl
from jax.experimental.pallas import tpu as pltpu
from jax.experimental.pallas import tpu_sc as plsc

N = 100000
E = 3200000
F = 16
K = 4
L = 3
G = 64
NEG_SLOPE = 0.01

NC = 2
NS = 16
NW = NC * NS
IDX_W = 128
CH = 2048
J = CH // IDX_W
EPT = E // NS
CPT = -(-EPT // CH)
EPTP = CPT * CH
ROWS_PT = EPTP // IDX_W

NP = 100352
HALF = NP // 2
STRIPE = HALF // NS
STR_CP = (2048, 1088)

BLK = 2048
TGRID = NP // BLK
PB = 1024
PG = NP // PB


def _sc_prop_body(u_hbm, row_hbm, col_hbm, w_hbm, part_hbm,
                  idxr0, idxc0, wbuf0, rows0, idxr1, idxc1, wbuf1, rows1,
                  acc, semg0, sems0, semg1, sems1):
    cc = lax.axis_index("c")
    ss = lax.axis_index("s")
    base = cc * HALF
    bufs = ((idxr0, idxc0, wbuf0, rows0, semg0, sems0),
            (idxr1, idxc1, wbuf1, rows1, semg1, sems1))

    for (_, idxc, _w, rows, _sg, _ss_) in bufs:
        def _z(i, _, rows=rows, idxc=idxc):
            rows[i, :] = jnp.zeros((16,), jnp.float32)
            if i is not None:
                pass
            return ()
        lax.fori_loop(0, CH, _z, (), unroll=False)
        def _zi(i, _, idxc=idxc):
            for t in range(8):
                idxc[i, pl.ds(t * 16, 16)] = jnp.zeros((16,), jnp.int32)
            return ()
        lax.fori_loop(0, J, _zi, (), unroll=False)
    off = 0
    for sz in STR_CP:
        pltpu.sync_copy(rows0.at[pl.ds(0, sz), :],
                        acc.at[pl.ds(ss * STRIPE + off, sz), :])
        off += sz
    plsc.subcore_barrier()
    for (_, idxc, _w, rows, _sg, sems) in bufs:
        for j in range(J):
            pltpu.async_copy(rows.at[pl.ds(j * IDX_W, IDX_W), :],
                             acc.at[idxc.at[j]], sems, add=True)

    HCPT = CPT // 2

    def _half_chunk(g, which):
        idxr, idxc, wbuf, rows, semg, sems = bufs[which]
        for j in range(J):
            pltpu.make_async_copy(rows.at[pl.ds(j * IDX_W, IDX_W), :],
                                  acc.at[idxc.at[j]], sems).wait()
        rbase = ss * ROWS_PT + g * J
        pltpu.sync_copy(row_hbm.at[pl.ds(rbase, J), :], idxr)
        pltpu.sync_copy(col_hbm.at[pl.ds(rbase, J), :], idxc)
        pltpu.sync_copy(w_hbm.at[pl.ds(rbase, J), :], wbuf)
        cps = [pltpu.async_copy(u_hbm.at[idxr.at[j]],
                                rows.at[pl.ds(j * IDX_W, IDX_W), :], semg)
               for j in range(J)]
        return cps

    def _finish_chunk(which, cps):
        idxr, idxc, wbuf, rows, semg, sems = bufs[which]
        for cp in cps:
            cp.wait()
        for j in range(J):
            def _grp(o, _, j=j, idxc=idxc, wbuf=wbuf, rows=rows):
                sl = pl.ds(o * 16, 16)
                lc = idxc[j, sl] - base
                m = (lc >= 0) & (lc < HALF)
                idxc[j, sl] = jnp.where(m, lc, 0)
                wrow = jnp.where(m, wbuf[j, sl], 0.0)
                f0 = j * IDX_W + o * 16
                for e in range(16):
                    wv = jnp.full((16,), wrow[e], jnp.float32)
                    rows[f0 + e, :] = rows[f0 + e, :] * wv
                return ()
            lax.fori_loop(0, 8, _grp, (), unroll=False)
        for j in range(J):
            pltpu.async_copy(rows.at[pl.ds(j * IDX_W, IDX_W), :],
                             acc.at[idxc.at[j]], sems, add=True)

    def _pair(gg, _):
        cps0 = _half_chunk(2 * gg, 0)
        cps1 = _half_chunk(2 * gg + 1, 1)
        _finish_chunk(0, cps0)
        _finish_chunk(1, cps1)
        return ()

    lax.fori_loop(0, HCPT, _pair, (), unroll=False)
    for (_, idxc, _w, rows, _sg, sems) in bufs:
        for j in range(J):
            pltpu.make_async_copy(rows.at[pl.ds(j * IDX_W, IDX_W), :],
                                  acc.at[idxc.at[j]], sems).wait()
    plsc.subcore_barrier()
    off = 0
    for sz in STR_CP:
        b = ss * STRIPE + off
        pltpu.sync_copy(acc.at[pl.ds(b, sz), :],
                        part_hbm.at[pl.ds(base + b, sz), :])
        off += sz


@jax.jit
def _sc_prop(u, rowp, colp, wp):
    mesh = plsc.VectorSubcoreMesh(core_axis_name="c", subcore_axis_name="s")
    return pl.kernel(
        _sc_prop_body,
        out_type=jax.ShapeDtypeStruct((NP, F), jnp.float32),
        mesh=mesh,
        compiler_params=pltpu.CompilerParams(use_tc_tiling_on_sc=False),
        scratch_types=[
            pltpu.VMEM((J, IDX_W), jnp.int32),
            pltpu.VMEM((J, IDX_W), jnp.int32),
            pltpu.VMEM((J, IDX_W), jnp.float32),
            pltpu.VMEM((CH, F), jnp.float32),
            pltpu.VMEM((J, IDX_W), jnp.int32),
            pltpu.VMEM((J, IDX_W), jnp.int32),
            pltpu.VMEM((J, IDX_W), jnp.float32),
            pltpu.VMEM((CH, F), jnp.float32),
            pltpu.VMEM_SHARED((HALF, F), jnp.float32),
            pltpu.SemaphoreType.DMA,
            pltpu.SemaphoreType.DMA,
            pltpu.SemaphoreType.DMA,
            pltpu.SemaphoreType.DMA,
        ],
    )(u, rowp, colp, wp)


def _dis_body(p_ref, o_ref):
    deg = p_ref[...]
    o_ref[...] = jnp.where(deg > 0, lax.rsqrt(jnp.where(deg > 0, deg, 1.0)),
                           0.0)


def _start_body(h_ref, dis_ref, wb_ref, u_ref, acc_ref):
    h = h_ref[...]
    u_ref[...] = dis_ref[...] * h
    acc_ref[...] = jnp.dot(h, wb_ref[...], preferred_element_type=jnp.float32)


def _mid_body(p_ref, dis_ref, acc_ref, wb_ref, u_ref, acco_ref):
    dis = dis_ref[...]
    cur = dis * p_ref[...]
    u_ref[...] = dis * cur
    acco_ref[...] = acc_ref[...] + jnp.dot(cur, wb_ref[...],
                                           preferred_element_type=jnp.float32)


def _end_body(p_ref, dis_ref, acc_ref, wb_ref, b_ref, h_ref):
    cur = dis_ref[...] * p_ref[...]
    t = acc_ref[...] + jnp.dot(cur, wb_ref[...],
                               preferred_element_type=jnp.float32) + b_ref[...]
    h_ref[...] = jnp.where(t >= 0, t, NEG_SLOPE * t)


def _pool_body(h_ref, bid_ref, wt_ref, b_ref, o_ref, macc, ccnt):
    step = pl.program_id(0)

    @pl.when(step == 0)
    def _():
        macc[...] = jnp.zeros_like(macc)
        ccnt[...] = jnp.zeros_like(ccnt)

    bid = bid_ref[0]
    seg = lax.broadcasted_iota(jnp.int32, (G, PB), 0)
    mask = (seg == bid).astype(jnp.float32)
    macc[...] += jnp.dot(mask, h_ref[...], preferred_element_type=jnp.float32)
    ccnt[...] += jnp.sum(mask, axis=1, keepdims=True)

    @pl.when(step == PG - 1)
    def _():
        pooled = macc[...] / jnp.maximum(ccnt[...], 1.0)
        o_ref[...] = (jnp.sum(pooled * wt_ref[...], axis=1, keepdims=True)
                      + b_ref[...])


_P_SPEC = pl.BlockSpec((BLK, F), lambda i: (i, 0))
_V_SPEC = pl.BlockSpec((BLK, F), lambda i: (i, 0))
_W_SPEC = pl.BlockSpec((F, F), lambda i: (0, 0))
_B_SPEC = pl.BlockSpec((1, F), lambda i: (0, 0))
_VSD = jax.ShapeDtypeStruct((NP, F), jnp.float32)


@jax.jit
def _tc_dis(p):
    return pl.pallas_call(
        _dis_body, grid=(TGRID,),
        in_specs=[_P_SPEC], out_specs=_V_SPEC, out_shape=_VSD,
    )(p)


@jax.jit
def _tc_start(h, dis, wb):
    return pl.pallas_call(
        _start_body, grid=(TGRID,),
        in_specs=[_V_SPEC, _V_SPEC, _W_SPEC],
        out_specs=[_V_SPEC, _V_SPEC], out_shape=[_VSD, _VSD],
    )(h, dis, wb)


@jax.jit
def _tc_mid(p, dis, acc, wb):
    return pl.pallas_call(
        _mid_body, grid=(TGRID,),
        in_specs=[_P_SPEC, _V_SPEC, _V_SPEC, _W_SPEC],
        out_specs=[_V_SPEC, _V_SPEC], out_shape=[_VSD, _VSD],
    )(p, dis, acc, wb)


@jax.jit
def _tc_end(p, dis, acc, wb, b2d):
    return pl.pallas_call(
        _end_body, grid=(TGRID,),
        in_specs=[_P_SPEC, _V_SPEC, _V_SPEC, _W_SPEC, _B_SPEC],
        out_specs=_V_SPEC, out_shape=_VSD,
    )(p, dis, acc, wb, b2d)


@jax.jit
def _tc_pool(hp, bidp, wt, b):
    return pl.pallas_call(
        _pool_body, grid=(PG,),
        in_specs=[pl.BlockSpec((PB, F), lambda i: (i, 0)),
                  pl.BlockSpec((1, 1, PB), lambda i: (i, 0, 0)),
                  pl.BlockSpec((1, F), lambda i: (0, 0)),
                  pl.BlockSpec((1, 1), lambda i: (0, 0))],
        out_specs=pl.BlockSpec((G, 1), lambda i: (0, 0)),
        out_shape=jax.ShapeDtypeStruct((G, 1), jnp.float32),
        scratch_shapes=[pltpu.VMEM((G, F), jnp.float32),
                        pltpu.VMEM((G, 1), jnp.float32)],
    )(hp, bidp, wt, b)


def kernel(x, edge_index, edge_attr, batch, conv_W, conv_b, lin_W, lin_b):
    w = edge_attr[:, 6]
    row, col = edge_index[0], edge_index[1]

    pad_i = jnp.zeros((NS, EPTP - EPT), jnp.int32)
    pad_f = jnp.zeros((NS, EPTP - EPT), jnp.float32)
    rowp = jnp.concatenate([row.reshape(NS, EPT), pad_i], 1).reshape(-1, IDX_W)
    colp = jnp.concatenate([col.reshape(NS, EPT), pad_i], 1).reshape(-1, IDX_W)
    wp = jnp.concatenate([w.reshape(NS, EPT), pad_f], 1).reshape(-1, IDX_W)

    xp = jnp.concatenate([x, jnp.zeros((NP - N, F), jnp.float32)], 0)

    ones = jnp.ones((NP, F), jnp.float32)
    dis = _tc_dis(_sc_prop(ones, rowp, colp, wp))

    h = xp
    for i in range(L):
        u, acc = _tc_start(h, dis, conv_W[i, 0])
        for k in range(1, K + 1):
            p = _sc_prop(u, rowp, colp, wp)
            if k < K:
                u, acc = _tc_mid(p, dis, acc, conv_W[i, k])
            else:
                h = _tc_end(p, dis, acc, conv_W[i, k],
                            conv_b[i].reshape(1, F))

    bidp = jnp.concatenate([batch, jnp.full((NP - N,), G, jnp.int32)],
                           0).reshape(PG, 1, PB)
    return _tc_pool(h, bidp, lin_W.reshape(1, F), lin_b.reshape(1, 1))


# speedup vs baseline: 1.0437x; 1.0437x over previous
import jax
import jax.numpy as jnp
from jax import lax
from jax.experimental import pallas as pl
from jax.experimental.pallas import tpu as pltpu
from jax.experimental.pallas import tpu_sc as plsc

N = 100000
E = 3200000
F = 16
K = 4
L = 3
G = 64
NEG_SLOPE = 0.01

NC = 2
NS = 16
NW = NC * NS
IDX_W = 128
CH = 2048
J = CH // IDX_W
EPT = E // NS
CPT = -(-EPT // CH)
EPTP = CPT * CH
ROWS_PT = EPTP // IDX_W

NP = 100352
HALF = NP // 2
STRIPE = HALF // NS
STR_CP = (2048, 1088)

BLK = 2048
TGRID = NP // BLK
PB = 1024
PG = NP // PB


def _sc_prop_body(u_hbm, row_hbm, col_hbm, w_hbm, part_hbm,
                  idxr, idxc, wbuf, rows, acc, semg, sems):
    cc = lax.axis_index("c")
    ss = lax.axis_index("s")
    base = cc * HALF

    def _z(i, _):
        rows[i, :] = jnp.zeros((16,), jnp.float32)
        return ()
    lax.fori_loop(0, CH, _z, (), unroll=False)
    off = 0
    for sz in STR_CP:
        pltpu.sync_copy(rows.at[pl.ds(0, sz), :],
                        acc.at[pl.ds(ss * STRIPE + off, sz), :])
        off += sz
    plsc.subcore_barrier()

    def _chunk(g, _):
        rbase = ss * ROWS_PT + g * J
        pltpu.sync_copy(row_hbm.at[pl.ds(rbase, J), :], idxr)
        pltpu.sync_copy(col_hbm.at[pl.ds(rbase, J), :], idxc)
        pltpu.sync_copy(w_hbm.at[pl.ds(rbase, J), :], wbuf)
        cps = [pltpu.async_copy(u_hbm.at[idxr.at[j]],
                                rows.at[pl.ds(j * IDX_W, IDX_W), :], semg)
               for j in range(J)]
        for cp in cps:
            cp.wait()
        for j in range(J):
            def _grp(o, _, j=j):
                sl = pl.ds(o * 16, 16)
                lc = idxc[j, sl] - base
                m = (lc >= 0) & (lc < HALF)
                idxc[j, sl] = jnp.where(m, lc, 0)
                wrow = jnp.where(m, wbuf[j, sl], 0.0)
                f0 = j * IDX_W + o * 16
                for e in range(16):
                    wv = jnp.full((16,), wrow[e], jnp.float32)
                    rows[f0 + e, :] = rows[f0 + e, :] * wv
                return ()
            lax.fori_loop(0, 8, _grp, (), unroll=False)
        scs = [pltpu.async_copy(rows.at[pl.ds(j * IDX_W, IDX_W), :],
                                acc.at[idxc.at[j]], sems, add=True)
               for j in range(J)]
        for cp in scs:
            cp.wait()
        return ()

    lax.fori_loop(0, CPT, _chunk, (), unroll=False)
    plsc.subcore_barrier()
    off = 0
    for sz in STR_CP:
        b = ss * STRIPE + off
        pltpu.sync_copy(acc.at[pl.ds(b, sz), :],
                        part_hbm.at[pl.ds(base + b, sz), :])
        off += sz


@jax.jit
def _sc_prop(u, rowp, colp, wp):
    mesh = plsc.VectorSubcoreMesh(core_axis_name="c", subcore_axis_name="s")
    return pl.kernel(
        _sc_prop_body,
        out_type=jax.ShapeDtypeStruct((NP, F), jnp.float32),
        mesh=mesh,
        compiler_params=pltpu.CompilerParams(use_tc_tiling_on_sc=False),
        scratch_types=[
            pltpu.VMEM((J, IDX_W), jnp.int32),
            pltpu.VMEM((J, IDX_W), jnp.int32),
            pltpu.VMEM((J, IDX_W), jnp.float32),
            pltpu.VMEM((CH, F), jnp.float32),
            pltpu.VMEM_SHARED((HALF, F), jnp.float32),
            pltpu.SemaphoreType.DMA,
            pltpu.SemaphoreType.DMA,
        ],
    )(u, rowp, colp, wp)


def _dis_body(p_ref, o_ref):
    deg = p_ref[...]
    o_ref[...] = jnp.where(deg > 0, lax.rsqrt(jnp.where(deg > 0, deg, 1.0)),
                           0.0)


def _start_body(h_ref, dis_ref, wb_ref, u_ref, acc_ref):
    h = h_ref[...]
    u_ref[...] = dis_ref[...] * h
    acc_ref[...] = jnp.dot(h, wb_ref[...], preferred_element_type=jnp.float32)


def _mid_body(p_ref, dis_ref, acc_ref, wb_ref, u_ref, acco_ref):
    dis = dis_ref[...]
    cur = dis * p_ref[...]
    u_ref[...] = dis * cur
    acco_ref[...] = acc_ref[...] + jnp.dot(cur, wb_ref[...],
                                           preferred_element_type=jnp.float32)


def _end_body(p_ref, dis_ref, acc_ref, wb_ref, b_ref, h_ref):
    cur = dis_ref[...] * p_ref[...]
    t = acc_ref[...] + jnp.dot(cur, wb_ref[...],
                               preferred_element_type=jnp.float32) + b_ref[...]
    h_ref[...] = jnp.where(t >= 0, t, NEG_SLOPE * t)


def _pool_body(h_ref, bid_ref, wt_ref, b_ref, o_ref, macc, ccnt):
    step = pl.program_id(0)

    @pl.when(step == 0)
    def _():
        macc[...] = jnp.zeros_like(macc)
        ccnt[...] = jnp.zeros_like(ccnt)

    bid = bid_ref[0]
    seg = lax.broadcasted_iota(jnp.int32, (G, PB), 0)
    mask = (seg == bid).astype(jnp.float32)
    macc[...] += jnp.dot(mask, h_ref[...], preferred_element_type=jnp.float32)
    ccnt[...] += jnp.sum(mask, axis=1, keepdims=True)

    @pl.when(step == PG - 1)
    def _():
        pooled = macc[...] / jnp.maximum(ccnt[...], 1.0)
        o_ref[...] = (jnp.sum(pooled * wt_ref[...], axis=1, keepdims=True)
                      + b_ref[...])


_P_SPEC = pl.BlockSpec((BLK, F), lambda i: (i, 0))
_V_SPEC = pl.BlockSpec((BLK, F), lambda i: (i, 0))
_W_SPEC = pl.BlockSpec((F, F), lambda i: (0, 0))
_B_SPEC = pl.BlockSpec((1, F), lambda i: (0, 0))
_VSD = jax.ShapeDtypeStruct((NP, F), jnp.float32)


@jax.jit
def _tc_dis(p):
    return pl.pallas_call(
        _dis_body, grid=(TGRID,),
        in_specs=[_P_SPEC], out_specs=_V_SPEC, out_shape=_VSD,
    )(p)


@jax.jit
def _tc_start(h, dis, wb):
    return pl.pallas_call(
        _start_body, grid=(TGRID,),
        in_specs=[_V_SPEC, _V_SPEC, _W_SPEC],
        out_specs=[_V_SPEC, _V_SPEC], out_shape=[_VSD, _VSD],
    )(h, dis, wb)


@jax.jit
def _tc_mid(p, dis, acc, wb):
    return pl.pallas_call(
        _mid_body, grid=(TGRID,),
        in_specs=[_P_SPEC, _V_SPEC, _V_SPEC, _W_SPEC],
        out_specs=[_V_SPEC, _V_SPEC], out_shape=[_VSD, _VSD],
    )(p, dis, acc, wb)


@jax.jit
def _tc_end(p, dis, acc, wb, b2d):
    return pl.pallas_call(
        _end_body, grid=(TGRID,),
        in_specs=[_P_SPEC, _V_SPEC, _V_SPEC, _W_SPEC, _B_SPEC],
        out_specs=_V_SPEC, out_shape=_VSD,
    )(p, dis, acc, wb, b2d)


@jax.jit
def _tc_pool(hp, bidp, wt, b):
    return pl.pallas_call(
        _pool_body, grid=(PG,),
        in_specs=[pl.BlockSpec((PB, F), lambda i: (i, 0)),
                  pl.BlockSpec((1, 1, PB), lambda i: (i, 0, 0)),
                  pl.BlockSpec((1, F), lambda i: (0, 0)),
                  pl.BlockSpec((1, 1), lambda i: (0, 0))],
        out_specs=pl.BlockSpec((G, 1), lambda i: (0, 0)),
        out_shape=jax.ShapeDtypeStruct((G, 1), jnp.float32),
        scratch_shapes=[pltpu.VMEM((G, F), jnp.float32),
                        pltpu.VMEM((G, 1), jnp.float32)],
    )(hp, bidp, wt, b)


def kernel(x, edge_index, edge_attr, batch, conv_W, conv_b, lin_W, lin_b):
    w = edge_attr[:, 6]
    row, col = edge_index[0], edge_index[1]

    pad_i = jnp.zeros((NS, EPTP - EPT), jnp.int32)
    pad_f = jnp.zeros((NS, EPTP - EPT), jnp.float32)
    rowp = jnp.concatenate([row.reshape(NS, EPT), pad_i], 1).reshape(-1, IDX_W)
    colp = jnp.concatenate([col.reshape(NS, EPT), pad_i], 1).reshape(-1, IDX_W)
    wp = jnp.concatenate([w.reshape(NS, EPT), pad_f], 1).reshape(-1, IDX_W)

    xp = jnp.concatenate([x, jnp.zeros((NP - N, F), jnp.float32)], 0)

    ones = jnp.ones((NP, F), jnp.float32)
    dis = _tc_dis(_sc_prop(ones, rowp, colp, wp))

    h = xp
    for i in range(L):
        u, acc = _tc_start(h, dis, conv_W[i, 0])
        for k in range(1, K + 1):
            p = _sc_prop(u, rowp, colp, wp)
            if k < K:
                u, acc = _tc_mid(p, dis, acc, conv_W[i, k])
            else:
                h = _tc_end(p, dis, acc, conv_W[i, k],
                            conv_b[i].reshape(1, F))

    bidp = jnp.concatenate([batch, jnp.full((NP - N,), G, jnp.int32)],
                           0).reshape(PG, 1, PB)
    return _tc_pool(h, bidp, lin_W.reshape(1, F), lin_b.reshape(1, 1))
